# Initial kernel scaffold; baseline (speedup 1.0000x reference)
import jax
import jax.numpy as jnp
from jax.experimental import pallas as pl


def kernel(x, edge_index, edge_weight, y, train_id, W1, b1, lw1, lb1, W2, b2, lw2, lb2, c1w, c1b, c2w, c2b):
    raise NotImplementedError("write your pallas kernel here")


# speedup vs baseline: 7.4101x; 7.4101x over previous
import functools

import jax
import jax.numpy as jnp
from jax import lax
from jax.experimental import pallas as pl
from jax.experimental.pallas import tpu as pltpu
from jax.experimental.pallas import tpu_sc as plsc

N = 10000
E = 320000
F_IN = 128
H = 128
C = 40
NT = 1000
CPAD = 128

RB = 1000
GRID = N // RB

NC = 2
NS = 16
LK = 80
EPW = E // NS
NCH = EPW // LK
NPAD = 10240
RPT = NPAD // NS
ZB = 128

NTP = 1024
RPW = NTP // (NC * NS)


def _support_body(x_ref, w1_ref, w2_ref, o_ref):
    xb = x_ref[...]
    o_ref[0] = jnp.dot(xb, w1_ref[...], preferred_element_type=jnp.float32)
    o_ref[1] = jnp.dot(xb, w2_ref[...], preferred_element_type=jnp.float32)


def _support(x, W1, W2):
    return pl.pallas_call(
        _support_body,
        grid=(GRID,),
        in_specs=[
            pl.BlockSpec((RB, F_IN), lambda i: (i, 0)),
            pl.BlockSpec((F_IN, H), lambda i: (0, 0)),
            pl.BlockSpec((F_IN, H), lambda i: (0, 0)),
        ],
        out_specs=pl.BlockSpec((2, RB, H), lambda i: (0, i, 0)),
        out_shape=jax.ShapeDtypeStruct((2, N, H), jnp.float32),
    )(x, W1, W2)


def _seg_body(sup_ref, src_ref, dst_ref, w_ref, out_ref,
              src_v, dst_v, w_v, rows_v, zbuf, acc_sh, sem):
    c = lax.axis_index("c")
    s = lax.axis_index("s")
    coff = c * N

    def zrow(i, carry):
        for j in range(H // 16):
            zbuf[i, pl.ds(j * 16, 16)] = jnp.zeros((16,), jnp.float32)
        return carry
    lax.fori_loop(0, ZB, zrow, None)
    for kk in range(RPT // ZB):
        pltpu.sync_copy(zbuf, acc_sh.at[pl.ds(s * RPT + kk * ZB, ZB)])
    plsc.subcore_barrier()

    def chunk(ch, carry):
        base = s * EPW + ch * LK
        pltpu.sync_copy(src_ref.at[pl.ds(base, LK)], src_v)
        pltpu.sync_copy(dst_ref.at[pl.ds(base, LK)], dst_v)
        pltpu.sync_copy(w_ref.at[pl.ds(base, LK)], w_v)

        def offb(g, cy):
            src_v[pl.ds(g * 16, 16)] = src_v[pl.ds(g * 16, 16)] + coff
            return cy
        lax.fori_loop(0, LK // 16, offb, None)

        pltpu.async_copy(sup_ref.at[src_v], rows_v, sem).wait()

        def scale_group(g, cy):
            wv = w_v[pl.ds(g * 16, 16)]
            def scale_lane(l, cz):
                idx = (jnp.zeros((16,), jnp.int32) + l).reshape(16, 1)
                ws = lax.gather(
                    wv, idx,
                    lax.GatherDimensionNumbers(
                        offset_dims=(), collapsed_slice_dims=(0,),
                        start_index_map=(0,)),
                    (1,), mode=lax.GatherScatterMode.PROMISE_IN_BOUNDS)
                i = g * 16 + l
                for j in range(H // 16):
                    rows_v[i, pl.ds(j * 16, 16)] = (
                        rows_v[i, pl.ds(j * 16, 16)] * ws)
                return cz
            lax.fori_loop(0, 16, scale_lane, None)
            return cy
        lax.fori_loop(0, LK // 16, scale_group, None)

        pltpu.sync_copy(rows_v, acc_sh.at[dst_v], add=True)
        return carry
    lax.fori_loop(0, NCH, chunk, None)
    plsc.subcore_barrier()

    pltpu.sync_copy(acc_sh.at[pl.ds(s * RPT, RPT)],
                    out_ref.at[c, pl.ds(s * RPT, RPT)])


def _segment_sum(support2n, src, dst, ew):
    mesh = plsc.VectorSubcoreMesh(core_axis_name="c", subcore_axis_name="s")
    f = functools.partial(
        pl.kernel, mesh=mesh,
        out_type=jax.ShapeDtypeStruct((2, NPAD, H), jnp.float32),
        scratch_types=[
            pltpu.VMEM((LK,), jnp.int32),
            pltpu.VMEM((LK,), jnp.int32),
            pltpu.VMEM((LK,), jnp.float32),
            pltpu.VMEM((LK, H), jnp.float32),
            pltpu.VMEM((ZB, H), jnp.float32),
            pltpu.VMEM_SHARED((NPAD, H), jnp.float32),
            pltpu.SemaphoreType.DMA,
        ],
    )(_seg_body)
    return f(support2n, src, dst, ew)


def _post_body(agg_ref, y_ref, b1_ref, lw1_ref, lb1_ref, b2_ref, lw2_ref,
               lb2_ref, c1w_ref, c1b_ref, c2w_ref, c2b_ref,
               z_ref, o_ref, opad_ref, z1n_ref, z2n_ref,
               g1_ref, g2_ref, st_ref):
    i = pl.program_id(0)
    h1 = jnp.maximum(agg_ref[0] + b1_ref[...], 0.0)
    z1 = jnp.dot(h1, lw1_ref[...], preferred_element_type=jnp.float32) \
        + lb1_ref[...]
    h2 = jnp.maximum(agg_ref[1] + b2_ref[...], 0.0)
    z2 = jnp.dot(h2, lw2_ref[...], preferred_element_type=jnp.float32) \
        + lb2_ref[...]
    z = (z1 + z2) * 0.5
    z_ref[...] = z

    t = jnp.dot(z, c1w_ref[...], preferred_element_type=jnp.float32) \
        + c1b_ref[...]
    logits = jnp.dot(t, c2w_ref[...], preferred_element_type=jnp.float32) \
        + c2b_ref[...]
    m = jnp.max(logits, axis=1, keepdims=True)
    sh = logits - m
    lse = jnp.log(jnp.sum(jnp.exp(sh), axis=1, keepdims=True))
    o = sh - lse
    o_ref[...] = o
    labf = y_ref[...].astype(jnp.float32)
    opad_ref[...] = jnp.concatenate(
        [o, labf, jnp.zeros((RB, CPAD - C - 1), jnp.float32)], axis=1)

    n1 = jnp.clip(jnp.sqrt(jnp.sum(z1 * z1, axis=1, keepdims=True)),
                  1e-12, None)
    n2 = jnp.clip(jnp.sqrt(jnp.sum(z2 * z2, axis=1, keepdims=True)),
                  1e-12, None)
    z1n = z1 / n1
    z2n = z2 / n2
    z1n_ref[...] = z1n
    z2n_ref[...] = z2n

    g1p = lax.dot_general(z1n, z1n, (((0,), (0,)), ((), ())),
                          preferred_element_type=jnp.float32)
    g2p = lax.dot_general(z2n, z2n, (((0,), (0,)), ((), ())),
                          preferred_element_type=jnp.float32)
    d = jnp.sum(z1n * z2n, axis=1)
    sd1 = jnp.sum((d - 1.0) ** 2)
    sd2 = jnp.sum(d * d)
    lane = lax.broadcasted_iota(jnp.int32, (1, 128), 1)
    stp = jnp.where(lane == 0, sd1, 0.0) + jnp.where(lane == 1, sd2, 0.0)

    @pl.when(i == 0)
    def _init():
        g1_ref[...] = g1p
        g2_ref[...] = g2p
        st_ref[...] = stp

    @pl.when(i != 0)
    def _acc():
        g1_ref[...] += g1p
        g2_ref[...] += g2p
        st_ref[...] += stp


def _post(agg, y, b1, lw1, lb1, b2, lw2, lb2, c1w, c1b, c2w, c2b):
    row = lambda v: v.reshape(1, -1)
    cst = lambda shp: pl.BlockSpec(shp, lambda i: (0, 0))
    return pl.pallas_call(
        _post_body,
        grid=(GRID,),
        in_specs=[
            pl.BlockSpec((2, RB, H), lambda i: (0, i, 0)),
            pl.BlockSpec((RB, 1), lambda i: (i, 0)),
            cst((1, H)), cst((H, H)), cst((1, H)),
            cst((1, H)), cst((H, H)), cst((1, H)),
            cst((H, H)), cst((1, H)), cst((H, C)), cst((1, C)),
        ],
        out_specs=[
            pl.BlockSpec((RB, H), lambda i: (i, 0)),
            pl.BlockSpec((RB, C), lambda i: (i, 0)),
            pl.BlockSpec((RB, CPAD), lambda i: (i, 0)),
            pl.BlockSpec((RB, H), lambda i: (i, 0)),
            pl.BlockSpec((RB, H), lambda i: (i, 0)),
            cst((H, H)), cst((H, H)), cst((1, 128)),
        ],
        out_shape=[
            jax.ShapeDtypeStruct((N, H), jnp.float32),
            jax.ShapeDtypeStruct((N, C), jnp.float32),
            jax.ShapeDtypeStruct((N, CPAD), jnp.float32),
            jax.ShapeDtypeStruct((N, H), jnp.float32),
            jax.ShapeDtypeStruct((N, H), jnp.float32),
            jax.ShapeDtypeStruct((H, H), jnp.float32),
            jax.ShapeDtypeStruct((H, H), jnp.float32),
            jax.ShapeDtypeStruct((1, 128), jnp.float32),
        ],
    )(agg, y.reshape(N, 1), row(b1), lw1, row(lb1), row(b2), lw2, row(lb2),
      c1w, row(c1b), c2w, row(c2b))


def _gather_body(z1n_ref, z2n_ref, opad_ref, tid_ref,
                 h1_ref, h2_ref, rows_ref,
                 tid_v, buf, buf64, sem):
    c = lax.axis_index("c")
    s = lax.axis_index("s")
    wid = s * NC + c
    base = wid * RPW
    pltpu.sync_copy(tid_ref.at[pl.ds(base, RPW)], tid_v)

    pltpu.async_copy(z1n_ref.at[tid_v], buf, sem).wait()
    pltpu.sync_copy(buf, h1_ref.at[pl.ds(base, RPW)])
    pltpu.async_copy(z2n_ref.at[tid_v], buf, sem).wait()
    pltpu.sync_copy(buf, h2_ref.at[pl.ds(base, RPW)])
    pltpu.async_copy(opad_ref.at[tid_v], buf64, sem).wait()
    pltpu.sync_copy(buf64, rows_ref.at[pl.ds(base, RPW)])


def _gather_train(z1n, z2n, opad, tid_pad):
    mesh = plsc.VectorSubcoreMesh(core_axis_name="c", subcore_axis_name="s")
    f = functools.partial(
        pl.kernel, mesh=mesh,
        out_type=(
            jax.ShapeDtypeStruct((NTP, H), jnp.float32),
            jax.ShapeDtypeStruct((NTP, H), jnp.float32),
            jax.ShapeDtypeStruct((NTP, CPAD), jnp.float32),
        ),
        scratch_types=[
            pltpu.VMEM((RPW,), jnp.int32),
            pltpu.VMEM((RPW, H), jnp.float32),
            pltpu.VMEM((RPW, CPAD), jnp.float32),
            pltpu.SemaphoreType.DMA,
        ],
    )(_gather_body)
    return f(z1n, z2n, opad, tid_pad)


def _final_body(h1_ref, h2_ref, rows_ref, laba_ref, labb_ref,
                g1_ref, g2_ref, st_ref, tot_ref):
    h1 = h1_ref[...]
    h2 = h2_ref[...]
    sim = lax.dot_general(h1, h2, (((1,), (1,)), ((), ())),
                          preferred_element_type=jnp.float32) * 2.0
    m = jnp.max(sim, axis=1, keepdims=True)
    lse = m + jnp.log(jnp.sum(jnp.exp(sim - m), axis=1, keepdims=True))
    logprob = sim - lse
    laba = laba_ref[...]
    labb = labb_ref[...]
    mask = (laba == labb).astype(jnp.float32)
    cl = -jnp.mean(jnp.sum(logprob * mask, axis=1)
                   / jnp.sum(mask, axis=1))

    lane = lax.broadcasted_iota(jnp.int32, (NT, CPAD), 1)
    onehot = (lane == laba).astype(jnp.float32)
    nll = -jnp.sum(rows_ref[...] * onehot) / NT

    tr = jnp.sum(g1_ref[...] * g2_ref[...])
    sd1 = st_ref[0, 0]
    sd2 = st_ref[0, 1]
    bt = sd1 / N + (tr - sd2) / (N * (N - 1))

    total = nll + 0.1 * bt + cl
    lane1 = lax.broadcasted_iota(jnp.int32, (1, 128), 1)
    tot_ref[...] = jnp.where(lane1 == 0, total, 0.0)


def _final(h1, h2, rows, laba, labb, g1, g2, st):
    full = lambda shp: pl.BlockSpec(shp, lambda: (0, 0))
    return pl.pallas_call(
        _final_body,
        grid=(),
        in_specs=[
            full((NT, H)), full((NT, H)), full((NT, CPAD)),
            full((NT, 1)), full((1, NT)),
            full((H, H)), full((H, H)), full((1, 128)),
        ],
        out_specs=full((1, 128)),
        out_shape=jax.ShapeDtypeStruct((1, 128), jnp.float32),
    )(h1, h2, rows, laba, labb, g1, g2, st)


def kernel(x, edge_index, edge_weight, y, train_id,
           W1, b1, lw1, lb1, W2, b2, lw2, lb2, c1w, c1b, c2w, c2b):
    src = edge_index[0]
    dst = edge_index[1]

    support = _support(x, W1, W2)
    agg = _segment_sum(support.reshape(2 * N, H), src, dst, edge_weight)
    z, o, opad, z1n, z2n, g1, g2, st = _post(
        agg, y, b1, lw1, lb1, b2, lw2, lb2, c1w, c1b, c2w, c2b)

    tid_pad = jnp.pad(train_id, (0, NTP - NT))
    h1, h2, rows = _gather_train(z1n, z2n, opad, tid_pad)
    h1 = h1[:NT]
    h2 = h2[:NT]
    rows = rows[:NT]
    labs = rows[:, C].astype(jnp.int32)

    tot = _final(h1, h2, rows, labs.reshape(NT, 1), labs.reshape(1, NT),
                 g1, g2, st)
    return (z, o, tot[0, 0])
